# baseline (speedup 1.0000x reference)
import functools

import jax
import jax.numpy as jnp
from jax import lax
from jax.experimental import pallas as pl
from jax.experimental.pallas import tpu as pltpu
from jax.experimental.pallas import tpu_sc as plsc

NUM_CORES = 2
NUM_SUBCORES = 16
NW = NUM_CORES * NUM_SUBCORES
BATCH = 16384
EMB = 64
BPW = BATCH // NW
CHUNK = 128
NCHUNK = BPW // CHUNK
LANES = 16
NGROUP = BPW // LANES


def _bpr_body(user_emb, item_emb, user_ids, pos_item_ids, neg_item_ids,
              pos_out, neg_out,
              uid_v, pid_v, nid_v, u_v, p_v, n_v, po_v, no_v, sem):
    wid = lax.axis_index("s") * NUM_CORES + lax.axis_index("c")
    base = wid * BPW

    for c in range(NCHUNK):
        off = base + c * CHUNK
        pltpu.sync_copy(user_ids.at[pl.ds(off, CHUNK)], uid_v.at[c])
        pltpu.sync_copy(pos_item_ids.at[pl.ds(off, CHUNK)], pid_v.at[c])
        pltpu.sync_copy(neg_item_ids.at[pl.ds(off, CHUNK)], nid_v.at[c])

    copies = []
    for c in range(NCHUNK):
        dst = pl.ds(c * CHUNK, CHUNK)
        copies.append(pltpu.async_copy(user_emb.at[uid_v.at[c]], u_v.at[dst], sem))
        copies.append(pltpu.async_copy(item_emb.at[pid_v.at[c]], p_v.at[dst], sem))
        copies.append(pltpu.async_copy(item_emb.at[nid_v.at[c]], n_v.at[dst], sem))
    for cp in copies:
        cp.wait()

    lanes = lax.iota(jnp.int32, LANES)

    def group_step(g, carry):
        rows = g * LANES + lanes
        acc_p = jnp.zeros((LANES,), jnp.float32)
        acc_n = jnp.zeros((LANES,), jnp.float32)
        for d in range(EMB):
            cols = jnp.full((LANES,), d, jnp.int32)
            uu = plsc.load_gather(u_v, [rows, cols])
            pp = plsc.load_gather(p_v, [rows, cols])
            nn = plsc.load_gather(n_v, [rows, cols])
            acc_p = acc_p + uu * pp
            acc_n = acc_n + uu * nn
        po_v[pl.ds(g * LANES, LANES)] = acc_p
        no_v[pl.ds(g * LANES, LANES)] = acc_n
        return carry

    lax.fori_loop(0, NGROUP, group_step, 0)

    pltpu.sync_copy(po_v, pos_out.at[pl.ds(base, BPW)])
    pltpu.sync_copy(no_v, neg_out.at[pl.ds(base, BPW)])


@jax.jit
def _bpr(user_emb, item_emb, user_ids, pos_item_ids, neg_item_ids):
    mesh = plsc.VectorSubcoreMesh(core_axis_name="c", subcore_axis_name="s")
    run = functools.partial(
        pl.kernel,
        out_type=(
            jax.ShapeDtypeStruct((BATCH,), jnp.float32),
            jax.ShapeDtypeStruct((BATCH,), jnp.float32),
        ),
        mesh=mesh,
        scratch_types=[
            pltpu.VMEM((NCHUNK, CHUNK), jnp.int32),
            pltpu.VMEM((NCHUNK, CHUNK), jnp.int32),
            pltpu.VMEM((NCHUNK, CHUNK), jnp.int32),
            pltpu.VMEM((BPW, EMB), jnp.float32),
            pltpu.VMEM((BPW, EMB), jnp.float32),
            pltpu.VMEM((BPW, EMB), jnp.float32),
            pltpu.VMEM((BPW,), jnp.float32),
            pltpu.VMEM((BPW,), jnp.float32),
            pltpu.SemaphoreType.DMA,
        ],
        compiler_params=pltpu.CompilerParams(
            needs_layout_passes=False, use_tc_tiling_on_sc=False),
    )(_bpr_body)
    return run(user_emb, item_emb, user_ids, pos_item_ids, neg_item_ids)


def kernel(user_emb, item_emb, user_ids, pos_item_ids, neg_item_ids):
    return _bpr(user_emb, item_emb,
                user_ids.astype(jnp.int32),
                pos_item_ids.astype(jnp.int32),
                neg_item_ids.astype(jnp.int32))


# speedup vs baseline: 1.5476x; 1.5476x over previous
import functools

import jax
import jax.numpy as jnp
from jax import lax
from jax.experimental import pallas as pl
from jax.experimental.pallas import tpu as pltpu
from jax.experimental.pallas import tpu_sc as plsc

NUM_CORES = 2
NUM_SUBCORES = 16
NW = NUM_CORES * NUM_SUBCORES
BATCH = 16384
EMB = 64
ROWPAD = 128
BPW = BATCH // NW
HALF = BPW // 2
CHUNK = 128
NCHUNK = HALF // CHUNK
LANES = 16
NGROUP = HALF // LANES


def _bpr_body(user_emb, item_emb, user_ids, pos_item_ids, neg_item_ids,
              pos_out, neg_out,
              uid_v, pid_v, nid_v,
              u_v, p_v, n_v, po_v, no_v, dummy_v, sem):
    wid = lax.axis_index("s") * NUM_CORES + lax.axis_index("c")
    base = wid * BPW

    pltpu.sync_copy(user_ids.at[pl.ds(base, BPW)], uid_v)
    pltpu.sync_copy(pos_item_ids.at[pl.ds(base, BPW)], pid_v)
    pltpu.sync_copy(neg_item_ids.at[pl.ds(base, BPW)], nid_v)

    lanes = lax.iota(jnp.int32, LANES)

    for h in range(2):
        hoff = h * HALF

        def fetch_step(g, carry):
            uu16 = uid_v[pl.ds(hoff + g * LANES, LANES)]
            pp16 = pid_v[pl.ds(hoff + g * LANES, LANES)]
            nn16 = nid_v[pl.ds(hoff + g * LANES, LANES)]
            for j in range(LANES):
                r = g * LANES + j
                pltpu.async_copy(user_emb.at[uu16[j]],
                                 u_v.at[r, pl.ds(0, EMB)], sem)
                pltpu.async_copy(item_emb.at[pp16[j]],
                                 p_v.at[r, pl.ds(0, EMB)], sem)
                pltpu.async_copy(item_emb.at[nn16[j]],
                                 n_v.at[r, pl.ds(0, EMB)], sem)
            return carry

        lax.fori_loop(0, NGROUP, fetch_step, 0)

        for _ in range(3):
            pltpu.make_async_copy(pos_out, dummy_v, sem).wait()

        def group_step(g, carry):
            rows = g * LANES + lanes
            acc_p = jnp.zeros((LANES,), jnp.float32)
            acc_n = jnp.zeros((LANES,), jnp.float32)
            for d in range(EMB):
                cols = jnp.full((LANES,), d, jnp.int32)
                uu = plsc.load_gather(u_v, [rows, cols])
                pp = plsc.load_gather(p_v, [rows, cols])
                nn = plsc.load_gather(n_v, [rows, cols])
                acc_p = acc_p + uu * pp
                acc_n = acc_n + uu * nn
            po_v[pl.ds(hoff + g * LANES, LANES)] = acc_p
            no_v[pl.ds(hoff + g * LANES, LANES)] = acc_n
            return carry

        lax.fori_loop(0, NGROUP, group_step, 0)

    pltpu.sync_copy(po_v, pos_out.at[pl.ds(base, BPW)])
    pltpu.sync_copy(no_v, neg_out.at[pl.ds(base, BPW)])


@jax.jit
def _bpr(user_emb, item_emb, user_ids, pos_item_ids, neg_item_ids):
    mesh = plsc.VectorSubcoreMesh(core_axis_name="c", subcore_axis_name="s")
    run = functools.partial(
        pl.kernel,
        out_type=(
            jax.ShapeDtypeStruct((BATCH,), jnp.float32),
            jax.ShapeDtypeStruct((BATCH,), jnp.float32),
        ),
        mesh=mesh,
        scratch_types=[
            pltpu.VMEM((BPW,), jnp.int32),
            pltpu.VMEM((BPW,), jnp.int32),
            pltpu.VMEM((BPW,), jnp.int32),
            pltpu.VMEM((HALF, ROWPAD), jnp.float32),
            pltpu.VMEM((HALF, ROWPAD), jnp.float32),
            pltpu.VMEM((HALF, ROWPAD), jnp.float32),
            pltpu.VMEM((BPW,), jnp.float32),
            pltpu.VMEM((BPW,), jnp.float32),
            pltpu.VMEM((BATCH,), jnp.float32),
            pltpu.SemaphoreType.DMA,
        ],
        compiler_params=pltpu.CompilerParams(needs_layout_passes=False),
    )(_bpr_body)
    return run(user_emb, item_emb, user_ids, pos_item_ids, neg_item_ids)


def kernel(user_emb, item_emb, user_ids, pos_item_ids, neg_item_ids):
    return _bpr(user_emb, item_emb,
                user_ids.astype(jnp.int32),
                pos_item_ids.astype(jnp.int32),
                neg_item_ids.astype(jnp.int32))
